# baseline (speedup 1.0000x reference)
import jax
import jax.numpy as jnp
from jax import lax
from jax.experimental import pallas as pl
from jax.experimental.pallas import tpu as pltpu
from jax.experimental.pallas import tpu_sc as plsc

N = 320000
G = 128
T = 16
S = 256
SCALE = 1000000.0

_INFO = plsc.get_sparse_core_info()
NC = _INFO.num_cores
NS = _INFO.num_subcores
NW = NC * NS
CHUNK = 128

SC_PER_W = 30
SC_CHUNKS = SC_PER_W * NW
SC_ROWS = SC_CHUNKS * CHUNK
B_TC = 1280
K_TC = (N - SC_ROWS) // B_TC

RING = 5
DIST = 2
NJ = SC_PER_W // RING

ROWS_PER_SUB = S // NS
NV = G // 16


def _sc_body(x_hbm, idx_hbm, out_hbm, accum_sh, accum_loc, ibuf_all,
             zbuf, idbuf, xbufs, g0, g1, g2, g3, g4):
    gsems = (g0, g1, g2, g3, g4)
    c = lax.axis_index("c")
    s = lax.axis_index("s")
    wid = s * NC + c

    fzero16 = jnp.zeros((16,), jnp.float32)
    for r in range(ROWS_PER_SUB):
        for j in range(NV):
            zbuf[r, pl.ds(j * 16, 16)] = fzero16
    pltpu.sync_copy(zbuf, accum_sh.at[pl.ds(s * ROWS_PER_SUB, ROWS_PER_SUB)])
    plsc.subcore_barrier()

    def zrow(r, carry):
        for j in range(NV):
            accum_loc[r, pl.ds(j * 16, 16)] = fzero16
        return carry

    lax.fori_loop(0, S, zrow, 0)

    lane = lax.iota(jnp.int32, 16)
    for h in range(2):
        for j in range(NV):
            idbuf[h, pl.ds(j * 16, 16)] = lane + (128 * h + 16 * j)

    start = wid * SC_PER_W

    pltpu.sync_copy(idx_hbm.at[pl.ds(start, SC_PER_W)], ibuf_all)

    def issue_gather(t, b):
        pltpu.async_copy(
            x_hbm.at[pl.ds((start + t) * CHUNK, CHUNK)], xbufs.at[b], gsems[b])

    def wait_gather(t, b):
        pltpu.make_async_copy(
            x_hbm.at[pl.ds((start + t) * CHUNK, CHUNK)], xbufs.at[b],
            gsems[b]).wait()

    def process_chunk(t, b, idx_row):
        lo = idx_row[pl.ds(0, 16)]
        hi = lo
        for k in range(1, NV):
            v = idx_row[pl.ds(k * 16, 16)]
            lo = jnp.minimum(lo, v)
            hi = jnp.maximum(hi, v)
        seg_lo = jnp.min(lo)
        seg_hi = jnp.max(hi)

        def local_reduce():
            def rows8(r8, accs):
                accs = list(accs)
                for rr in range(8):
                    for j in range(NV):
                        accs[j] = accs[j] + xbufs[b, r8 * 8 + rr,
                                                 pl.ds(j * 16, 16)]
                return tuple(accs)

            accs = lax.fori_loop(
                0, CHUNK // 8, rows8,
                tuple(jnp.zeros((16,), jnp.float32) for _ in range(NV)))
            for j in range(NV):
                plsc.addupdate(accum_loc.at[seg_lo, pl.ds(j * 16, 16)],
                               accs[j])

        def stream_scatter():
            pltpu.sync_copy(xbufs.at[b], accum_sh.at[idx_row], add=True)

        lax.cond(seg_lo == seg_hi, local_reduce, stream_scatter)

    for b in range(DIST):
        issue_gather(b, b)

    def step(j, carry):
        for b in range(RING):
            t = j * RING + b
            bp = (b + DIST) % RING
            tp = t + DIST

            @pl.when(tp < SC_PER_W)
            def _(tp=tp, bp=bp):
                issue_gather(tp, bp)

            wait_gather(t, b)
            process_chunk(t, b, ibuf_all.at[t])
        return carry

    lax.fori_loop(0, NJ, step, 0)

    pltpu.sync_copy(accum_loc.at[pl.ds(0, 128)], accum_sh.at[idbuf.at[0]],
                    add=True)
    pltpu.sync_copy(accum_loc.at[pl.ds(128, 128)], accum_sh.at[idbuf.at[1]],
                    add=True)
    plsc.subcore_barrier()

    pltpu.sync_copy(
        accum_sh.at[pl.ds(s * ROWS_PER_SUB, ROWS_PER_SUB)],
        out_hbm.at[c, pl.ds(s * ROWS_PER_SUB, ROWS_PER_SUB)],
    )


_sc_segment_sum = pl.kernel(
    _sc_body,
    out_type=jax.ShapeDtypeStruct((NC, S, G), jnp.float32),
    mesh=plsc.VectorSubcoreMesh(core_axis_name="c", subcore_axis_name="s"),
    scratch_types=[
        pltpu.VMEM_SHARED((S, G), jnp.float32),
        pltpu.VMEM((S, G), jnp.float32),
        pltpu.VMEM((SC_PER_W, CHUNK), jnp.int32),
        pltpu.VMEM((ROWS_PER_SUB, G), jnp.float32),
        pltpu.VMEM((2, CHUNK), jnp.int32),
        pltpu.VMEM((RING, CHUNK, G), jnp.float32),
    ] + [pltpu.SemaphoreType.DMA] * RING,
    compiler_params=pltpu.CompilerParams(use_tc_tiling_on_sc=False,
                                         needs_layout_passes=False),
)


W_TC = 32


def _tc_partial_body(idx_ref, x_ref, out_ref, acc_ref):
    i = pl.program_id(0)

    @pl.when(i == 0)
    def _():
        acc_ref[...] = jnp.zeros((S, G), jnp.float32)

    idx = idx_ref[0, 0, :]
    seg_lo = jnp.min(idx)
    seg_hi = jnp.max(idx)
    wbase = jnp.minimum(seg_lo & ~7, S - W_TC)
    xb = x_ref[...].astype(jnp.bfloat16)

    @pl.when(seg_hi - wbase < W_TC)
    def _():
        iota_w = lax.broadcasted_iota(jnp.int32, (W_TC, B_TC), 0) + wbase
        onehot = (iota_w == idx[None, :]).astype(jnp.bfloat16)
        part = jnp.dot(onehot, xb, preferred_element_type=jnp.float32)
        acc_ref[pl.ds(wbase, W_TC), :] += part

    @pl.when(seg_hi - wbase >= W_TC)
    def _():
        iota_s = lax.broadcasted_iota(jnp.int32, (S, B_TC), 0)
        onehot = (iota_s == idx[None, :]).astype(jnp.bfloat16)
        part = jnp.dot(onehot, xb, preferred_element_type=jnp.float32)
        acc_ref[...] += part

    @pl.when(i == K_TC - 1)
    def _():
        out_ref[...] = acc_ref[...]


TC_BLK0 = SC_ROWS // B_TC


def _tc_partial(idx3d, x_full):
    return pl.pallas_call(
        _tc_partial_body,
        grid=(K_TC,),
        in_specs=[
            pl.BlockSpec((1, 1, B_TC), lambda i: (TC_BLK0 + i, 0, 0)),
            pl.BlockSpec((B_TC, G), lambda i: (TC_BLK0 + i, 0)),
        ],
        out_specs=pl.BlockSpec((S, G), lambda i: (0, 0)),
        out_shape=jax.ShapeDtypeStruct((S, G), jnp.float32),
        scratch_shapes=[pltpu.VMEM((S, G), jnp.float32)],
    )(idx3d, x_full)


def _tc_body(p_ref, ptc_ref, w_ref, ilr_ref, xb_ref):
    xb = p_ref[0] + p_ref[1] + ptc_ref[...]
    row_sums = jnp.sum(xb, axis=1, keepdims=True)
    xbn = xb * (SCALE / jnp.maximum(row_sums, 1e-12))
    xb_ref[...] = xbn
    ilr_ref[...] = lax.dot_general(
        xbn, w_ref[...], (((1,), (1,)), ((), ())),
        preferred_element_type=jnp.float32,
    )


def _tc_finish(partials, p_tc, W):
    return pl.pallas_call(
        _tc_body,
        out_shape=(
            jax.ShapeDtypeStruct((S, T), jnp.float32),
            jax.ShapeDtypeStruct((S, G), jnp.float32),
        ),
    )(partials, p_tc, W)


@jax.jit
def kernel(X_batch, batch_idx, W):
    idx32 = batch_idx.astype(jnp.int32)
    idx2d = idx32.reshape(N // CHUNK, CHUNK)
    idx3d = idx32.reshape(N // B_TC, 1, B_TC)
    partials = _sc_segment_sum(X_batch, idx2d)
    p_tc = _tc_partial(idx3d, X_batch)
    ilr_y, X_bulk = _tc_finish(partials, p_tc, W)
    return (ilr_y, X_bulk)


# speedup vs baseline: 1.0051x; 1.0051x over previous
import jax
import jax.numpy as jnp
from jax import lax
from jax.experimental import pallas as pl
from jax.experimental.pallas import tpu as pltpu
from jax.experimental.pallas import tpu_sc as plsc

N = 320000
G = 128
T = 16
S = 256
SCALE = 1000000.0

_INFO = plsc.get_sparse_core_info()
NC = _INFO.num_cores
NS = _INFO.num_subcores
NW = NC * NS
CHUNK = 128

SC_PER_W = 30
SC_CHUNKS = SC_PER_W * NW
SC_ROWS = SC_CHUNKS * CHUNK
B_TC = 1280
K_TC = (N - SC_ROWS) // B_TC

RING = 5
DIST = 2
NJ = SC_PER_W // RING

ROWS_PER_SUB = S // NS
NV = G // 16


def _sc_body(x_hbm, idx_hbm, out_hbm, accum_sh, accum_loc, ibuf_all,
             zbuf, idbuf, xbufs, g0, g1, g2, g3, g4):
    gsems = (g0, g1, g2, g3, g4)
    c = lax.axis_index("c")
    s = lax.axis_index("s")
    wid = s * NC + c

    fzero16 = jnp.zeros((16,), jnp.float32)
    for r in range(ROWS_PER_SUB):
        for j in range(NV):
            zbuf[r, pl.ds(j * 16, 16)] = fzero16
    pltpu.sync_copy(zbuf, accum_sh.at[pl.ds(s * ROWS_PER_SUB, ROWS_PER_SUB)])
    plsc.subcore_barrier()

    def zrow(r, carry):
        for j in range(NV):
            accum_loc[r, pl.ds(j * 16, 16)] = fzero16
        return carry

    lax.fori_loop(0, S, zrow, 0)

    lane = lax.iota(jnp.int32, 16)
    for h in range(2):
        for j in range(NV):
            idbuf[h, pl.ds(j * 16, 16)] = lane + (128 * h + 16 * j)

    start = wid * SC_PER_W

    pltpu.sync_copy(idx_hbm.at[pl.ds(start, SC_PER_W)], ibuf_all)

    def issue_gather(t, b):
        pltpu.async_copy(
            x_hbm.at[pl.ds((start + t) * CHUNK, CHUNK)], xbufs.at[b], gsems[b])

    def wait_gather(t, b):
        pltpu.make_async_copy(
            x_hbm.at[pl.ds((start + t) * CHUNK, CHUNK)], xbufs.at[b],
            gsems[b]).wait()

    def process_chunk(t, b, idx_row):
        lo = idx_row[pl.ds(0, 16)]
        hi = lo
        for k in range(1, NV):
            v = idx_row[pl.ds(k * 16, 16)]
            lo = jnp.minimum(lo, v)
            hi = jnp.maximum(hi, v)
        seg_lo = jnp.min(lo)
        seg_hi = jnp.max(hi)

        def local_reduce():
            def rows8(r8, accs):
                accs = list(accs)
                for rr in range(8):
                    for j in range(NV):
                        accs[j] = accs[j] + xbufs[b, r8 * 8 + rr,
                                                 pl.ds(j * 16, 16)]
                return tuple(accs)

            accs = lax.fori_loop(
                0, CHUNK // 8, rows8,
                tuple(jnp.zeros((16,), jnp.float32) for _ in range(NV)))
            for j in range(NV):
                plsc.addupdate(accum_loc.at[seg_lo, pl.ds(j * 16, 16)],
                               accs[j])

        def stream_scatter():
            pltpu.sync_copy(xbufs.at[b], accum_sh.at[idx_row], add=True)

        lax.cond(seg_lo == seg_hi, local_reduce, stream_scatter)

    for b in range(DIST):
        issue_gather(b, b)

    def step(j, carry):
        for b in range(RING):
            t = j * RING + b
            bp = (b + DIST) % RING
            tp = t + DIST

            @pl.when(tp < SC_PER_W)
            def _(tp=tp, bp=bp):
                issue_gather(tp, bp)

            wait_gather(t, b)
            process_chunk(t, b, ibuf_all.at[t])
        return carry

    lax.fori_loop(0, NJ, step, 0)

    pltpu.sync_copy(accum_loc.at[pl.ds(0, 128)], accum_sh.at[idbuf.at[0]],
                    add=True)
    pltpu.sync_copy(accum_loc.at[pl.ds(128, 128)], accum_sh.at[idbuf.at[1]],
                    add=True)
    plsc.subcore_barrier()

    pltpu.sync_copy(
        accum_sh.at[pl.ds(s * ROWS_PER_SUB, ROWS_PER_SUB)],
        out_hbm.at[c, pl.ds(s * ROWS_PER_SUB, ROWS_PER_SUB)],
    )


_sc_segment_sum = pl.kernel(
    _sc_body,
    out_type=jax.ShapeDtypeStruct((NC, S, G), jnp.float32),
    mesh=plsc.VectorSubcoreMesh(core_axis_name="c", subcore_axis_name="s"),
    scratch_types=[
        pltpu.VMEM_SHARED((S, G), jnp.float32),
        pltpu.VMEM((S, G), jnp.float32),
        pltpu.VMEM((SC_PER_W, CHUNK), jnp.int32),
        pltpu.VMEM((ROWS_PER_SUB, G), jnp.float32),
        pltpu.VMEM((2, CHUNK), jnp.int32),
        pltpu.VMEM((RING, CHUNK, G), jnp.float32),
    ] + [pltpu.SemaphoreType.DMA] * RING,
    compiler_params=pltpu.CompilerParams(use_tc_tiling_on_sc=False,
                                         needs_layout_passes=False),
)


W_TC = 32


def _tc_partial_body(idx_ref, x_ref, out_ref, acc_ref):
    i = pl.program_id(0)

    @pl.when(i == 0)
    def _():
        acc_ref[...] = jnp.zeros((S, G), jnp.float32)

    idx = idx_ref[0, 0, :]
    seg_lo = jnp.min(idx)
    seg_hi = jnp.max(idx)
    wbase = jnp.minimum(seg_lo & ~7, S - W_TC)
    xb = x_ref[...].astype(jnp.bfloat16)

    def windowed():
        iota_w = lax.broadcasted_iota(jnp.int32, (W_TC, B_TC), 0) + wbase
        onehot = (iota_w == idx[None, :]).astype(jnp.bfloat16)
        part = jnp.dot(onehot, xb, preferred_element_type=jnp.float32)
        acc_ref[pl.ds(wbase, W_TC), :] += part

    def full():
        iota_s = lax.broadcasted_iota(jnp.int32, (S, B_TC), 0)
        onehot = (iota_s == idx[None, :]).astype(jnp.bfloat16)
        part = jnp.dot(onehot, xb, preferred_element_type=jnp.float32)
        acc_ref[...] += part

    lax.cond(seg_hi - wbase < W_TC, windowed, full)

    @pl.when(i == K_TC - 1)
    def _():
        out_ref[...] = acc_ref[...]


TC_BLK0 = SC_ROWS // B_TC


def _tc_partial(idx3d, x_full):
    return pl.pallas_call(
        _tc_partial_body,
        grid=(K_TC,),
        in_specs=[
            pl.BlockSpec((1, 1, B_TC), lambda i: (TC_BLK0 + i, 0, 0)),
            pl.BlockSpec((B_TC, G), lambda i: (TC_BLK0 + i, 0)),
        ],
        out_specs=pl.BlockSpec((S, G), lambda i: (0, 0)),
        out_shape=jax.ShapeDtypeStruct((S, G), jnp.float32),
        scratch_shapes=[pltpu.VMEM((S, G), jnp.float32)],
    )(idx3d, x_full)


def _tc_body(p_ref, ptc_ref, w_ref, ilr_ref, xb_ref):
    xb = p_ref[0] + p_ref[1] + ptc_ref[...]
    row_sums = jnp.sum(xb, axis=1, keepdims=True)
    xbn = xb * (SCALE / jnp.maximum(row_sums, 1e-12))
    xb_ref[...] = xbn
    ilr_ref[...] = lax.dot_general(
        xbn, w_ref[...], (((1,), (1,)), ((), ())),
        preferred_element_type=jnp.float32,
    )


def _tc_finish(partials, p_tc, W):
    return pl.pallas_call(
        _tc_body,
        out_shape=(
            jax.ShapeDtypeStruct((S, T), jnp.float32),
            jax.ShapeDtypeStruct((S, G), jnp.float32),
        ),
    )(partials, p_tc, W)


@jax.jit
def kernel(X_batch, batch_idx, W):
    idx32 = batch_idx.astype(jnp.int32)
    idx2d = idx32.reshape(N // CHUNK, CHUNK)
    idx3d = idx32.reshape(N // B_TC, 1, B_TC)
    partials = _sc_segment_sum(X_batch, idx2d)
    p_tc = _tc_partial(idx3d, X_batch)
    ilr_y, X_bulk = _tc_finish(partials, p_tc, W)
    return (ilr_y, X_bulk)


# speedup vs baseline: 1.3751x; 1.3682x over previous
import jax
import jax.numpy as jnp
from jax import lax
from jax.experimental import pallas as pl
from jax.experimental.pallas import tpu as pltpu
from jax.experimental.pallas import tpu_sc as plsc

N = 320000
G = 128
T = 16
S = 256
SCALE = 1000000.0

_INFO = plsc.get_sparse_core_info()
NC = _INFO.num_cores
NS = _INFO.num_subcores
NW = NC * NS
CHUNK = 128
NCHUNKS = N // CHUNK
BASE_PER_W = NCHUNKS // NW
EXTRA = NCHUNKS - BASE_PER_W * NW

RING = 3
DIST = 2
NJ = BASE_PER_W // RING

ROWS_PER_SUB = S // NS
NV = G // 16


def _sc_body(x_hbm, idx_hbm, out_hbm, accum_sh, accum_loc, ibuf_all, ibuf_x,
             zbuf, idbuf, xbufs, g0, g1, g2, s0, s1, s2):
    gsems = (g0, g1, g2)
    ssems = (s0, s1, s2)
    c = lax.axis_index("c")
    s = lax.axis_index("s")
    wid = s * NC + c

    fzero16 = jnp.zeros((16,), jnp.float32)
    for r in range(ROWS_PER_SUB):
        for j in range(NV):
            zbuf[r, pl.ds(j * 16, 16)] = fzero16
    pltpu.sync_copy(zbuf, accum_sh.at[pl.ds(s * ROWS_PER_SUB, ROWS_PER_SUB)])
    plsc.subcore_barrier()

    def zrow(r, carry):
        for j in range(NV):
            accum_loc[r, pl.ds(j * 16, 16)] = fzero16
        return carry

    lax.fori_loop(0, S, zrow, 0)

    lane = lax.iota(jnp.int32, 16)
    for h in range(2):
        for j in range(NV):
            idbuf[h, pl.ds(j * 16, 16)] = lane + (128 * h + 16 * j)

    start = wid * BASE_PER_W + jnp.minimum(wid, EXTRA)

    pltpu.sync_copy(idx_hbm.at[pl.ds(start, BASE_PER_W)], ibuf_all)

    @pl.when(wid < EXTRA)
    def _():
        pltpu.sync_copy(idx_hbm.at[start + BASE_PER_W], ibuf_x)

    def issue_gather(t, b):
        pltpu.async_copy(
            x_hbm.at[pl.ds((start + t) * CHUNK, CHUNK)], xbufs.at[b], gsems[b])

    def wait_gather(t, b):
        pltpu.make_async_copy(
            x_hbm.at[pl.ds((start + t) * CHUNK, CHUNK)], xbufs.at[b],
            gsems[b]).wait()

    def wait_scatter(t, b):
        pltpu.make_async_copy(
            xbufs.at[b], accum_sh.at[ibuf_all.at[t]], ssems[b]).wait()

    def process_chunk(t, b, idx_row, route_mod=0):
        lo = idx_row[pl.ds(0, 16)]
        hi = lo
        for k in range(1, NV):
            v = idx_row[pl.ds(k * 16, 16)]
            lo = jnp.minimum(lo, v)
            hi = jnp.maximum(hi, v)
        seg_lo = jnp.min(lo)
        seg_hi = jnp.max(hi)

        def local_reduce():
            def rows8(r8, accs):
                accs = list(accs)
                for rr in range(8):
                    for j in range(NV):
                        accs[j] = accs[j] + xbufs[b, r8 * 8 + rr,
                                                 pl.ds(j * 16, 16)]
                return tuple(accs)

            accs = lax.fori_loop(
                0, CHUNK // 8, rows8,
                tuple(jnp.zeros((16,), jnp.float32) for _ in range(NV)))
            for j in range(NV):
                plsc.addupdate(accum_loc.at[seg_lo, pl.ds(j * 16, 16)],
                               accs[j])

        def stream_scatter():
            pltpu.async_copy(xbufs.at[b], accum_sh.at[idx_row], ssems[b],
                             add=True)

        uniform = seg_lo == seg_hi
        if route_mod:
            uniform = jnp.logical_and(uniform, (t % route_mod) != 0)
        lax.cond(uniform, local_reduce, stream_scatter)
        return 1 - uniform.astype(jnp.int32)

    for b in range(DIST):
        issue_gather(b, b)

    def step(j, flags):
        flags = list(flags)
        for b in range(RING):
            t = j * RING + b
            bp = (b + DIST) % RING
            tp = t + DIST

            @pl.when(jnp.logical_and(flags[bp] == 1, tp >= RING))
            def _(tp=tp, bp=bp):
                wait_scatter(tp - RING, bp)
            flags[bp] = 0

            @pl.when(tp < BASE_PER_W)
            def _(tp=tp, bp=bp):
                issue_gather(tp, bp)

            wait_gather(t, b)
            flags[b] = process_chunk(t, b, ibuf_all.at[t], route_mod=4)
        return tuple(flags)

    zi = jnp.int32(0)
    flags = lax.fori_loop(0, NJ, step, (zi, zi, zi))

    for b in range(RING):
        t_last = BASE_PER_W - RING + ((b - BASE_PER_W) % RING)

        @pl.when(flags[b] == 1)
        def _(t_last=t_last, b=b):
            wait_scatter(t_last, b)

    @pl.when(wid < EXTRA)
    def _():
        pltpu.sync_copy(
            x_hbm.at[pl.ds((start + BASE_PER_W) * CHUNK, CHUNK)], xbufs.at[0])
        was_stream = process_chunk(0, 0, ibuf_x)

        @pl.when(was_stream == 1)
        def _():
            pltpu.make_async_copy(
                xbufs.at[0], accum_sh.at[ibuf_x], ssems[0]).wait()

    pltpu.sync_copy(accum_loc.at[pl.ds(0, 128)], accum_sh.at[idbuf.at[0]],
                    add=True)
    pltpu.sync_copy(accum_loc.at[pl.ds(128, 128)], accum_sh.at[idbuf.at[1]],
                    add=True)
    plsc.subcore_barrier()

    pltpu.sync_copy(
        accum_sh.at[pl.ds(s * ROWS_PER_SUB, ROWS_PER_SUB)],
        out_hbm.at[c, pl.ds(s * ROWS_PER_SUB, ROWS_PER_SUB)],
    )


_sc_segment_sum = pl.kernel(
    _sc_body,
    out_type=jax.ShapeDtypeStruct((NC, S, G), jnp.float32),
    mesh=plsc.VectorSubcoreMesh(core_axis_name="c", subcore_axis_name="s"),
    scratch_types=[
        pltpu.VMEM_SHARED((S, G), jnp.float32),
        pltpu.VMEM((S, G), jnp.float32),
        pltpu.VMEM((BASE_PER_W, CHUNK), jnp.int32),
        pltpu.VMEM((CHUNK,), jnp.int32),
        pltpu.VMEM((ROWS_PER_SUB, G), jnp.float32),
        pltpu.VMEM((2, CHUNK), jnp.int32),
        pltpu.VMEM((RING, CHUNK, G), jnp.float32),
    ] + [pltpu.SemaphoreType.DMA] * (2 * RING),
    compiler_params=pltpu.CompilerParams(use_tc_tiling_on_sc=False,
                                         needs_layout_passes=False),
)


def _tc_body(p_ref, w_ref, ilr_ref, xb_ref):
    xb = p_ref[0] + p_ref[1]
    row_sums = jnp.sum(xb, axis=1, keepdims=True)
    xbn = xb * (SCALE / jnp.maximum(row_sums, 1e-12))
    xb_ref[...] = xbn
    ilr_ref[...] = lax.dot_general(
        xbn, w_ref[...], (((1,), (1,)), ((), ())),
        preferred_element_type=jnp.float32,
    )


def _tc_finish(partials, W):
    return pl.pallas_call(
        _tc_body,
        out_shape=(
            jax.ShapeDtypeStruct((S, T), jnp.float32),
            jax.ShapeDtypeStruct((S, G), jnp.float32),
        ),
    )(partials, W)


@jax.jit
def kernel(X_batch, batch_idx, W):
    idx2d = batch_idx.astype(jnp.int32).reshape(NCHUNKS, CHUNK)
    partials = _sc_segment_sum(X_batch, idx2d)
    ilr_y, X_bulk = _tc_finish(partials, W)
    return (ilr_y, X_bulk)


# speedup vs baseline: 1.4618x; 1.0630x over previous
import jax
import jax.numpy as jnp
from jax import lax
from jax.experimental import pallas as pl
from jax.experimental.pallas import tpu as pltpu
from jax.experimental.pallas import tpu_sc as plsc

N = 320000
G = 128
T = 16
S = 256
SCALE = 1000000.0

_INFO = plsc.get_sparse_core_info()
NC = _INFO.num_cores
NS = _INFO.num_subcores
NW = NC * NS
CHUNK = 128
NCHUNKS = N // CHUNK
BASE_PER_W = NCHUNKS // NW
EXTRA = NCHUNKS - BASE_PER_W * NW

GROUP = 2
NGROUPS = BASE_PER_W // GROUP
RING = 2
NJ = (NGROUPS - 1) // RING

ROWS_PER_SUB = S // NS
NV = G // 16


def _sc_body(x_hbm, idx_hbm, out_hbm, accum_sh, accum_loc, ibuf_all, ibuf_x,
             zbuf, idbuf, xbufs, g0, g1):
    gsems = (g0, g1)
    c = lax.axis_index("c")
    s = lax.axis_index("s")
    wid = s * NC + c

    fzero16 = jnp.zeros((16,), jnp.float32)
    for r in range(ROWS_PER_SUB):
        for j in range(NV):
            zbuf[r, pl.ds(j * 16, 16)] = fzero16
    pltpu.sync_copy(zbuf, accum_sh.at[pl.ds(s * ROWS_PER_SUB, ROWS_PER_SUB)])
    plsc.subcore_barrier()

    def zrow(r, carry):
        for j in range(NV):
            accum_loc[r, pl.ds(j * 16, 16)] = fzero16
        return carry

    lax.fori_loop(0, S, zrow, 0)

    lane = lax.iota(jnp.int32, 16)
    for h in range(2):
        for j in range(NV):
            idbuf[h, pl.ds(j * 16, 16)] = lane + (128 * h + 16 * j)

    start = wid * BASE_PER_W + jnp.minimum(wid, EXTRA)

    pltpu.sync_copy(idx_hbm.at[pl.ds(start, BASE_PER_W)], ibuf_all)

    @pl.when(wid < EXTRA)
    def _():
        pltpu.sync_copy(idx_hbm.at[start + BASE_PER_W], ibuf_x)

    def issue_gather(g, b):
        pltpu.async_copy(
            x_hbm.at[pl.ds((start + g * GROUP) * CHUNK, GROUP * CHUNK)],
            xbufs.at[b], gsems[b])

    def wait_gather(g, b):
        pltpu.make_async_copy(
            x_hbm.at[pl.ds((start + g * GROUP) * CHUNK, GROUP * CHUNK)],
            xbufs.at[b], gsems[b]).wait()

    def process_chunk(b, off, idx_row):
        lo = idx_row[pl.ds(0, 16)]
        hi = lo
        for k in range(1, NV):
            v = idx_row[pl.ds(k * 16, 16)]
            lo = jnp.minimum(lo, v)
            hi = jnp.maximum(hi, v)
        seg_lo = jnp.min(lo)
        seg_hi = jnp.max(hi)

        def local_reduce():
            def rows8(r8, accs):
                accs = list(accs)
                for rr in range(8):
                    for j in range(NV):
                        accs[j] = accs[j] + xbufs[b, off + r8 * 8 + rr,
                                                 pl.ds(j * 16, 16)]
                return tuple(accs)

            accs = lax.fori_loop(
                0, CHUNK // 8, rows8,
                tuple(jnp.zeros((16,), jnp.float32) for _ in range(NV)))
            for j in range(NV):
                plsc.addupdate(accum_loc.at[seg_lo, pl.ds(j * 16, 16)],
                               accs[j])

        def stream_scatter():
            pltpu.sync_copy(xbufs.at[b, pl.ds(off, CHUNK)],
                            accum_sh.at[idx_row], add=True)

        lax.cond(seg_lo == seg_hi, local_reduce, stream_scatter)

    def process_group(g, b):
        for half in range(GROUP):
            process_chunk(b, half * CHUNK, ibuf_all.at[g * GROUP + half])

    issue_gather(0, 0)

    def step(j, carry):
        for b in range(RING):
            g = j * RING + b
            issue_gather(g + 1, (b + 1) % RING)
            wait_gather(g, b)
            process_group(g, b)
        return carry

    lax.fori_loop(0, NJ, step, 0)

    g_last = NGROUPS - 1
    wait_gather(g_last, g_last % RING)
    process_group(g_last, g_last % RING)

    @pl.when(wid < EXTRA)
    def _():
        pltpu.sync_copy(
            x_hbm.at[pl.ds((start + BASE_PER_W) * CHUNK, CHUNK)],
            xbufs.at[0, pl.ds(0, CHUNK)])
        process_chunk(0, 0, ibuf_x)

    pltpu.sync_copy(accum_loc.at[pl.ds(0, 128)], accum_sh.at[idbuf.at[0]],
                    add=True)
    pltpu.sync_copy(accum_loc.at[pl.ds(128, 128)], accum_sh.at[idbuf.at[1]],
                    add=True)
    plsc.subcore_barrier()

    pltpu.sync_copy(
        accum_sh.at[pl.ds(s * ROWS_PER_SUB, ROWS_PER_SUB)],
        out_hbm.at[c, pl.ds(s * ROWS_PER_SUB, ROWS_PER_SUB)],
    )


_sc_segment_sum = pl.kernel(
    _sc_body,
    out_type=jax.ShapeDtypeStruct((NC, S, G), jnp.float32),
    mesh=plsc.VectorSubcoreMesh(core_axis_name="c", subcore_axis_name="s"),
    scratch_types=[
        pltpu.VMEM_SHARED((S, G), jnp.float32),
        pltpu.VMEM((S, G), jnp.float32),
        pltpu.VMEM((BASE_PER_W, CHUNK), jnp.int32),
        pltpu.VMEM((CHUNK,), jnp.int32),
        pltpu.VMEM((ROWS_PER_SUB, G), jnp.float32),
        pltpu.VMEM((2, CHUNK), jnp.int32),
        pltpu.VMEM((RING, GROUP * CHUNK, G), jnp.float32),
    ] + [pltpu.SemaphoreType.DMA] * RING,
    compiler_params=pltpu.CompilerParams(use_tc_tiling_on_sc=False,
                                         needs_layout_passes=False),
)


def _tc_body(p_ref, w_ref, ilr_ref, xb_ref):
    xb = p_ref[0] + p_ref[1]
    row_sums = jnp.sum(xb, axis=1, keepdims=True)
    xbn = xb * (SCALE / jnp.maximum(row_sums, 1e-12))
    xb_ref[...] = xbn
    ilr_ref[...] = lax.dot_general(
        xbn, w_ref[...], (((1,), (1,)), ((), ())),
        preferred_element_type=jnp.float32,
    )


def _tc_finish(partials, W):
    return pl.pallas_call(
        _tc_body,
        out_shape=(
            jax.ShapeDtypeStruct((S, T), jnp.float32),
            jax.ShapeDtypeStruct((S, G), jnp.float32),
        ),
    )(partials, W)


@jax.jit
def kernel(X_batch, batch_idx, W):
    idx2d = batch_idx.astype(jnp.int32).reshape(NCHUNKS, CHUNK)
    partials = _sc_segment_sum(X_batch, idx2d)
    ilr_y, X_bulk = _tc_finish(partials, W)
    return (ilr_y, X_bulk)


# speedup vs baseline: 1.6423x; 1.1235x over previous
import jax
import jax.numpy as jnp
from jax import lax
from jax.experimental import pallas as pl
from jax.experimental.pallas import tpu as pltpu
from jax.experimental.pallas import tpu_sc as plsc

N = 320000
G = 128
T = 16
S = 256
SCALE = 1000000.0

_INFO = plsc.get_sparse_core_info()
NC = _INFO.num_cores
NS = _INFO.num_subcores
NW = NC * NS
CHUNK = 128
NCHUNKS = N // CHUNK
BASE_PER_W = NCHUNKS // NW
EXTRA = NCHUNKS - BASE_PER_W * NW

RING = 4
DIST = 3
NJ = BASE_PER_W // RING
PEEL = BASE_PER_W - NJ * RING

ROWS_PER_SUB = S // NS
NV = G // 16


def _sc_body(x_hbm, idx_hbm, out_hbm, accum_sh, accum_loc, ibuf_all, ibuf_x,
             zbuf, idbuf, xbufs, g0, g1, g2, g3):
    gsems = (g0, g1, g2, g3)
    c = lax.axis_index("c")
    s = lax.axis_index("s")
    wid = s * NC + c

    fzero16 = jnp.zeros((16,), jnp.float32)
    for r in range(ROWS_PER_SUB):
        for j in range(NV):
            zbuf[r, pl.ds(j * 16, 16)] = fzero16
    pltpu.sync_copy(zbuf, accum_sh.at[pl.ds(s * ROWS_PER_SUB, ROWS_PER_SUB)])
    plsc.subcore_barrier()

    def zrow(r, carry):
        for j in range(NV):
            accum_loc[r, pl.ds(j * 16, 16)] = fzero16
        return carry

    lax.fori_loop(0, S, zrow, 0)

    lane = lax.iota(jnp.int32, 16)
    for h in range(2):
        for j in range(NV):
            idbuf[h, pl.ds(j * 16, 16)] = lane + (128 * h + 16 * j)

    start = wid * BASE_PER_W + jnp.minimum(wid, EXTRA)

    pltpu.sync_copy(idx_hbm.at[pl.ds(start, BASE_PER_W)], ibuf_all)

    @pl.when(wid < EXTRA)
    def _():
        pltpu.sync_copy(idx_hbm.at[start + BASE_PER_W], ibuf_x)

    def issue_gather(t, b):
        pltpu.async_copy(
            x_hbm.at[pl.ds((start + t) * CHUNK, CHUNK)], xbufs.at[b], gsems[b])

    def wait_gather(t, b):
        pltpu.make_async_copy(
            x_hbm.at[pl.ds((start + t) * CHUNK, CHUNK)], xbufs.at[b],
            gsems[b]).wait()

    def process_chunk(t, b, idx_row):
        lo = idx_row[pl.ds(0, 16)]
        hi = lo
        for k in range(1, NV):
            v = idx_row[pl.ds(k * 16, 16)]
            lo = jnp.minimum(lo, v)
            hi = jnp.maximum(hi, v)
        seg_lo = jnp.min(lo)
        seg_hi = jnp.max(hi)

        def local_reduce():
            def rows8(r8, accs):
                accs = list(accs)
                for rr in range(8):
                    for j in range(NV):
                        accs[j] = accs[j] + xbufs[b, r8 * 8 + rr,
                                                 pl.ds(j * 16, 16)]
                return tuple(accs)

            accs = lax.fori_loop(
                0, CHUNK // 8, rows8,
                tuple(jnp.zeros((16,), jnp.float32) for _ in range(NV)))
            for j in range(NV):
                plsc.addupdate(accum_loc.at[seg_lo, pl.ds(j * 16, 16)],
                               accs[j])

        def stream_scatter():
            pltpu.sync_copy(xbufs.at[b], accum_sh.at[idx_row], add=True)

        lax.cond(seg_lo == seg_hi, local_reduce, stream_scatter)

    for b in range(DIST):
        issue_gather(b, b)

    def step(j, carry):
        for b in range(RING):
            t = j * RING + b
            bp = (b + DIST) % RING
            tp = t + DIST

            @pl.when(tp < BASE_PER_W)
            def _(tp=tp, bp=bp):
                issue_gather(tp, bp)

            wait_gather(t, b)
            process_chunk(t, b, ibuf_all.at[t])
        return carry

    lax.fori_loop(0, NJ, step, 0)

    for k in range(PEEL):
        t = NJ * RING + k
        wait_gather(t, t % RING)
        process_chunk(t, t % RING, ibuf_all.at[t])

    @pl.when(wid < EXTRA)
    def _():
        pltpu.sync_copy(
            x_hbm.at[pl.ds((start + BASE_PER_W) * CHUNK, CHUNK)], xbufs.at[0])
        process_chunk(0, 0, ibuf_x)

    pltpu.sync_copy(accum_loc.at[pl.ds(0, 128)], accum_sh.at[idbuf.at[0]],
                    add=True)
    pltpu.sync_copy(accum_loc.at[pl.ds(128, 128)], accum_sh.at[idbuf.at[1]],
                    add=True)
    plsc.subcore_barrier()

    pltpu.sync_copy(
        accum_sh.at[pl.ds(s * ROWS_PER_SUB, ROWS_PER_SUB)],
        out_hbm.at[c, pl.ds(s * ROWS_PER_SUB, ROWS_PER_SUB)],
    )


_sc_segment_sum = pl.kernel(
    _sc_body,
    out_type=jax.ShapeDtypeStruct((NC, S, G), jnp.float32),
    mesh=plsc.VectorSubcoreMesh(core_axis_name="c", subcore_axis_name="s"),
    scratch_types=[
        pltpu.VMEM_SHARED((S, G), jnp.float32),
        pltpu.VMEM((S, G), jnp.float32),
        pltpu.VMEM((BASE_PER_W, CHUNK), jnp.int32),
        pltpu.VMEM((CHUNK,), jnp.int32),
        pltpu.VMEM((ROWS_PER_SUB, G), jnp.float32),
        pltpu.VMEM((2, CHUNK), jnp.int32),
        pltpu.VMEM((RING, CHUNK, G), jnp.float32),
    ] + [pltpu.SemaphoreType.DMA] * RING,
    name="sc_segment_sum",
    compiler_params=pltpu.CompilerParams(use_tc_tiling_on_sc=False,
                                         needs_layout_passes=False),
)


def _tc_body(p_ref, w_ref, ilr_ref, xb_ref):
    xb = p_ref[0] + p_ref[1]
    row_sums = jnp.sum(xb, axis=1, keepdims=True)
    xbn = xb * (SCALE / jnp.maximum(row_sums, 1e-12))
    xb_ref[...] = xbn
    ilr_ref[...] = lax.dot_general(
        xbn, w_ref[...], (((1,), (1,)), ((), ())),
        preferred_element_type=jnp.float32,
    )


def _tc_finish(partials, W):
    return pl.pallas_call(
        _tc_body,
        out_shape=(
            jax.ShapeDtypeStruct((S, T), jnp.float32),
            jax.ShapeDtypeStruct((S, G), jnp.float32),
        ),
    )(partials, W)


@jax.jit
def kernel(X_batch, batch_idx, W):
    idx2d = batch_idx.astype(jnp.int32).reshape(NCHUNKS, CHUNK)
    partials = _sc_segment_sum(X_batch, idx2d)
    ilr_y, X_bulk = _tc_finish(partials, W)
    return (ilr_y, X_bulk)


# speedup vs baseline: 1.6616x; 1.0118x over previous
import jax
import jax.numpy as jnp
from jax import lax
from jax.experimental import pallas as pl
from jax.experimental.pallas import tpu as pltpu
from jax.experimental.pallas import tpu_sc as plsc

N = 320000
G = 128
T = 16
S = 256
SCALE = 1000000.0

_INFO = plsc.get_sparse_core_info()
NC = _INFO.num_cores
NS = _INFO.num_subcores
NW = NC * NS
CHUNK = 128
NCHUNKS = N // CHUNK
BASE_PER_W = NCHUNKS // NW
EXTRA = NCHUNKS - BASE_PER_W * NW

RING = 5
DIST = 4
NJ = BASE_PER_W // RING
PEEL = BASE_PER_W - NJ * RING

ROWS_PER_SUB = S // NS
NV = G // 16


def _sc_body(x_hbm, idx_hbm, out_hbm, accum_sh, accum_loc, ibuf_all, ibuf_x,
             zbuf, idbuf, xbufs, g0, g1, g2, g3, g4):
    gsems = (g0, g1, g2, g3, g4)
    c = lax.axis_index("c")
    s = lax.axis_index("s")
    wid = s * NC + c

    fzero16 = jnp.zeros((16,), jnp.float32)
    for r in range(ROWS_PER_SUB):
        for j in range(NV):
            zbuf[r, pl.ds(j * 16, 16)] = fzero16
    pltpu.sync_copy(zbuf, accum_sh.at[pl.ds(s * ROWS_PER_SUB, ROWS_PER_SUB)])
    plsc.subcore_barrier()

    def zrow(r, carry):
        for j in range(NV):
            accum_loc[r, pl.ds(j * 16, 16)] = fzero16
        return carry

    lax.fori_loop(0, S, zrow, 0)

    lane = lax.iota(jnp.int32, 16)
    for h in range(2):
        for j in range(NV):
            idbuf[h, pl.ds(j * 16, 16)] = lane + (128 * h + 16 * j)

    start = wid * BASE_PER_W + jnp.minimum(wid, EXTRA)

    pltpu.sync_copy(idx_hbm.at[pl.ds(start, BASE_PER_W)], ibuf_all)

    @pl.when(wid < EXTRA)
    def _():
        pltpu.sync_copy(idx_hbm.at[start + BASE_PER_W], ibuf_x)

    def issue_gather(t, b):
        pltpu.async_copy(
            x_hbm.at[pl.ds((start + t) * CHUNK, CHUNK)], xbufs.at[b], gsems[b])

    def wait_gather(t, b):
        pltpu.make_async_copy(
            x_hbm.at[pl.ds((start + t) * CHUNK, CHUNK)], xbufs.at[b],
            gsems[b]).wait()

    def process_chunk(t, b, idx_row):
        lo = idx_row[pl.ds(0, 16)]
        hi = lo
        for k in range(1, NV):
            v = idx_row[pl.ds(k * 16, 16)]
            lo = jnp.minimum(lo, v)
            hi = jnp.maximum(hi, v)
        seg_lo = jnp.min(lo)
        seg_hi = jnp.max(hi)

        def local_reduce():
            def rows8(r8, accs):
                accs = list(accs)
                for rr in range(8):
                    for j in range(NV):
                        accs[j] = accs[j] + xbufs[b, r8 * 8 + rr,
                                                 pl.ds(j * 16, 16)]
                return tuple(accs)

            accs = lax.fori_loop(
                0, CHUNK // 8, rows8,
                tuple(jnp.zeros((16,), jnp.float32) for _ in range(NV)))
            for j in range(NV):
                plsc.addupdate(accum_loc.at[seg_lo, pl.ds(j * 16, 16)],
                               accs[j])

        def stream_scatter():
            pltpu.sync_copy(xbufs.at[b], accum_sh.at[idx_row], add=True)

        lax.cond(seg_lo == seg_hi, local_reduce, stream_scatter)

    for b in range(DIST):
        issue_gather(b, b)

    def step(j, carry):
        for b in range(RING):
            t = j * RING + b
            bp = (b + DIST) % RING
            tp = t + DIST

            @pl.when(tp < BASE_PER_W)
            def _(tp=tp, bp=bp):
                issue_gather(tp, bp)

            wait_gather(t, b)
            process_chunk(t, b, ibuf_all.at[t])
        return carry

    lax.fori_loop(0, NJ, step, 0)

    for k in range(PEEL):
        t = NJ * RING + k
        wait_gather(t, t % RING)
        process_chunk(t, t % RING, ibuf_all.at[t])

    @pl.when(wid < EXTRA)
    def _():
        pltpu.sync_copy(
            x_hbm.at[pl.ds((start + BASE_PER_W) * CHUNK, CHUNK)], xbufs.at[0])
        process_chunk(0, 0, ibuf_x)

    pltpu.sync_copy(accum_loc.at[pl.ds(0, 128)], accum_sh.at[idbuf.at[0]],
                    add=True)
    pltpu.sync_copy(accum_loc.at[pl.ds(128, 128)], accum_sh.at[idbuf.at[1]],
                    add=True)
    plsc.subcore_barrier()

    pltpu.sync_copy(
        accum_sh.at[pl.ds(s * ROWS_PER_SUB, ROWS_PER_SUB)],
        out_hbm.at[c, pl.ds(s * ROWS_PER_SUB, ROWS_PER_SUB)],
    )


_sc_segment_sum = pl.kernel(
    _sc_body,
    out_type=jax.ShapeDtypeStruct((NC, S, G), jnp.float32),
    mesh=plsc.VectorSubcoreMesh(core_axis_name="c", subcore_axis_name="s"),
    scratch_types=[
        pltpu.VMEM_SHARED((S, G), jnp.float32),
        pltpu.VMEM((S, G), jnp.float32),
        pltpu.VMEM((BASE_PER_W, CHUNK), jnp.int32),
        pltpu.VMEM((CHUNK,), jnp.int32),
        pltpu.VMEM((ROWS_PER_SUB, G), jnp.float32),
        pltpu.VMEM((2, CHUNK), jnp.int32),
        pltpu.VMEM((RING, CHUNK, G), jnp.float32),
    ] + [pltpu.SemaphoreType.DMA] * RING,
    name="sc_segment_sum",
    compiler_params=pltpu.CompilerParams(use_tc_tiling_on_sc=False,
                                         needs_layout_passes=False),
)


def _tc_body(p_ref, w_ref, ilr_ref, xb_ref):
    xb = p_ref[0] + p_ref[1]
    row_sums = jnp.sum(xb, axis=1, keepdims=True)
    xbn = xb * (SCALE / jnp.maximum(row_sums, 1e-12))
    xb_ref[...] = xbn
    ilr_ref[...] = lax.dot_general(
        xbn, w_ref[...], (((1,), (1,)), ((), ())),
        preferred_element_type=jnp.float32,
    )


def _tc_finish(partials, W):
    return pl.pallas_call(
        _tc_body,
        out_shape=(
            jax.ShapeDtypeStruct((S, T), jnp.float32),
            jax.ShapeDtypeStruct((S, G), jnp.float32),
        ),
    )(partials, W)


@jax.jit
def kernel(X_batch, batch_idx, W):
    idx2d = batch_idx.astype(jnp.int32).reshape(NCHUNKS, CHUNK)
    partials = _sc_segment_sum(X_batch, idx2d)
    ilr_y, X_bulk = _tc_finish(partials, W)
    return (ilr_y, X_bulk)
